# baseline (speedup 1.0000x reference)
import functools

import jax
import jax.numpy as jnp
from jax import lax
from jax.experimental import pallas as pl
from jax.experimental.pallas import tpu as pltpu
from jax.experimental.pallas import tpu_sc as plsc

N = 10000
E = 320000
D = 128

NC = 2
NS = 16
NW = NC * NS
EPW = E // NW
CHUNK = 80
NCH = EPW // CHUNK
assert NCH * CHUNK == EPW, (NCH, CHUNK)
RPT = 624
TAIL = N - NS * RPT


NBUF = 4
NOUT = NCH // NBUF
REM = NCH - NOUT * NBUF


def _sc_scatter(h, src3, dst3):
    mesh = plsc.VectorSubcoreMesh(core_axis_name="c", subcore_axis_name="s")

    @functools.partial(
        pl.kernel,
        out_type=jax.ShapeDtypeStruct((NC, N, D), jnp.float32),
        mesh=mesh,
        scratch_types=[
            [pltpu.VMEM((CHUNK,), jnp.int32) for _ in range(NBUF)],
            [pltpu.VMEM((CHUNK,), jnp.int32) for _ in range(NBUF)],
            [pltpu.VMEM((CHUNK, D), jnp.float32) for _ in range(NBUF)],
            pltpu.VMEM_SHARED((N, D), jnp.float32),
            pltpu.SemaphoreType.DMA((NBUF,)),
            pltpu.SemaphoreType.DMA((NBUF,)),
            pltpu.SemaphoreType.DMA((NBUF,)),
            pltpu.SemaphoreType.DMA((NBUF,)),
        ],
    )
    def k(h_hbm, src_hbm, dst_hbm, out_hbm, srcb, dstb, rows, acc_sh,
          sem_is, sem_id, sem_g, sem_s):
        c = lax.axis_index("c")
        s = lax.axis_index("s")
        wid = s * NC + c

        pltpu.sync_copy(h_hbm.at[pl.ds(s * RPT, RPT)], acc_sh.at[pl.ds(s * RPT, RPT)])

        @pl.when(s == NS - 1)
        def _():
            pltpu.sync_copy(h_hbm.at[pl.ds(NS * RPT, TAIL)],
                            acc_sh.at[pl.ds(NS * RPT, TAIL)])

        plsc.subcore_barrier()

        for r in range(REM):
            pltpu.sync_copy(src_hbm.at[wid, r], srcb[0])
            pltpu.sync_copy(dst_hbm.at[wid, r], dstb[0])
            pltpu.async_copy(h_hbm.at[srcb[0]], rows[0], sem_g.at[0]).wait()
            pltpu.async_copy(rows[0], acc_sh.at[dstb[0]], sem_s.at[0],
                             add=True).wait()

        for b in range(NBUF):
            pltpu.async_copy(src_hbm.at[wid, REM + b], srcb[b], sem_is.at[b])
            pltpu.async_copy(dst_hbm.at[wid, REM + b], dstb[b], sem_id.at[b])
        for b in range(NBUF):
            pltpu.make_async_copy(src_hbm.at[wid, REM + b], srcb[b],
                                  sem_is.at[b]).wait()
            pltpu.async_copy(h_hbm.at[srcb[b]], rows[b], sem_g.at[b])

        def outer(jo, carry):
            for b in range(NBUF):
                j = REM + jo * NBUF + b
                pltpu.make_async_copy(h_hbm.at[srcb[b]], rows[b],
                                      sem_g.at[b]).wait()
                pltpu.make_async_copy(dst_hbm.at[wid, j], dstb[b],
                                      sem_id.at[b]).wait()
                pltpu.async_copy(rows[b], acc_sh.at[dstb[b]],
                                 sem_s.at[b], add=True)

            @pl.when(jo < NOUT - 1)
            def _():
                jn0 = REM + (jo + 1) * NBUF
                for b in range(NBUF):
                    pltpu.async_copy(src_hbm.at[wid, jn0 + b], srcb[b],
                                     sem_is.at[b])
                for b in range(NBUF):
                    pltpu.make_async_copy(rows[b], acc_sh.at[dstb[b]],
                                          sem_s.at[b]).wait()
                    pltpu.async_copy(dst_hbm.at[wid, jn0 + b], dstb[b],
                                     sem_id.at[b])
                for b in range(NBUF):
                    pltpu.make_async_copy(src_hbm.at[wid, jn0 + b], srcb[b],
                                          sem_is.at[b]).wait()
                    pltpu.async_copy(h_hbm.at[srcb[b]], rows[b], sem_g.at[b])

            @pl.when(jo == NOUT - 1)
            def _():
                for b in range(NBUF):
                    pltpu.make_async_copy(rows[b], acc_sh.at[dstb[b]],
                                          sem_s.at[b]).wait()

            return carry

        lax.fori_loop(0, NOUT, outer, 0)

        plsc.subcore_barrier()

        pltpu.sync_copy(acc_sh.at[pl.ds(s * RPT, RPT)],
                        out_hbm.at[c, pl.ds(s * RPT, RPT)])

        @pl.when(s == NS - 1)
        def _():
            pltpu.sync_copy(acc_sh.at[pl.ds(NS * RPT, TAIL)],
                            out_hbm.at[c, pl.ds(NS * RPT, TAIL)])

    return k(h, src3, dst3)


def _mlp_chain(hprev, p0, p1, Ws, bs, flags, rows):
    grid = (N // rows,)
    nw = len(Ws)

    def body(*refs):
        x_ref, p0_ref, p1_ref = refs[0], refs[1], refs[2]
        w_refs = refs[3:3 + nw]
        b_refs = refs[3 + nw:3 + 2 * nw]
        o_ref = refs[-1]
        hloc = p0_ref[...] + p1_ref[...] - x_ref[...]
        kk = 0
        for g in range(nw):
            for i in range(w_refs[g].shape[0]):
                hloc = jnp.dot(hloc, w_refs[g][i],
                               preferred_element_type=jnp.float32)
                hloc = hloc + b_refs[g][i]
                if flags[kk]:
                    hloc = jnp.maximum(hloc, 0.0)
                kk += 1
        o_ref[...] = hloc

    row_spec = pl.BlockSpec((rows, D), lambda i: (i, 0))
    w_specs = [pl.BlockSpec(W.shape, lambda i: (0, 0, 0)) for W in Ws]
    b_specs = [pl.BlockSpec(b.shape, lambda i: (0, 0, 0)) for b in bs]
    return pl.pallas_call(
        body,
        grid=grid,
        in_specs=[row_spec, row_spec, row_spec] + w_specs + b_specs,
        out_specs=row_spec,
        out_shape=jax.ShapeDtypeStruct((N, D), jnp.float32),
    )(hprev, p0, p1, *Ws, *bs)


def kernel(x, edge_index, batch, conv_W, conv_b, lin_W, lin_b):
    src = edge_index[0].reshape(NW, NCH, CHUNK)
    dst = edge_index[1].reshape(NW, NCH, CHUNK)
    x = x.astype(jnp.float32)

    parts = _sc_scatter(x, src, dst)
    h = _mlp_chain(x, parts[0], parts[1],
                   [conv_W[0]], [conv_b[0].reshape(2, 1, D)],
                   flags=(True, True), rows=5000)

    parts = _sc_scatter(h, src, dst)
    flags = (True, False) + (True,) * 8 + (True,) * 7 + (False,)
    out = _mlp_chain(h, parts[0], parts[1],
                     [conv_W[1], lin_W.reshape(16, D, D)],
                     [conv_b[1].reshape(2, 1, D), lin_b.reshape(16, 1, D)],
                     flags=flags, rows=5000)
    return out


# speedup vs baseline: 1.0335x; 1.0335x over previous
import functools

import jax
import jax.numpy as jnp
from jax import lax
from jax.experimental import pallas as pl
from jax.experimental.pallas import tpu as pltpu
from jax.experimental.pallas import tpu_sc as plsc

N = 10000
E = 320000
D = 128

NC = 2
NS = 16
NW = NC * NS
EPW = E // NW
CHUNK = 80
NCH = EPW // CHUNK
assert NCH * CHUNK == EPW, (NCH, CHUNK)
RPT = 624
TAIL = N - NS * RPT


NBUF = 4
NOUT = NCH // NBUF
REM = NCH - NOUT * NBUF


def _sc_scatter(h, src3, dst3):
    mesh = plsc.VectorSubcoreMesh(core_axis_name="c", subcore_axis_name="s")

    @functools.partial(
        pl.kernel,
        out_type=jax.ShapeDtypeStruct((NC, N, D), jnp.float32),
        mesh=mesh,
        scratch_types=[
            [pltpu.VMEM((CHUNK,), jnp.int32) for _ in range(NBUF)],
            [pltpu.VMEM((CHUNK,), jnp.int32) for _ in range(NBUF)],
            [pltpu.VMEM((CHUNK, D), jnp.float32) for _ in range(NBUF)],
            pltpu.VMEM_SHARED((N, D), jnp.float32),
            pltpu.SemaphoreType.DMA((NBUF,)),
            pltpu.SemaphoreType.DMA((NBUF,)),
            pltpu.SemaphoreType.DMA((NBUF,)),
            pltpu.SemaphoreType.DMA((NBUF,)),
        ],
    )
    def k(h_hbm, src_hbm, dst_hbm, out_hbm, srcb, dstb, rows, acc_sh,
          sem_is, sem_id, sem_g, sem_s):
        c = lax.axis_index("c")
        s = lax.axis_index("s")
        wid = s * NC + c

        pltpu.sync_copy(h_hbm.at[pl.ds(s * RPT, RPT)], acc_sh.at[pl.ds(s * RPT, RPT)])

        @pl.when(s == NS - 1)
        def _():
            pltpu.sync_copy(h_hbm.at[pl.ds(NS * RPT, TAIL)],
                            acc_sh.at[pl.ds(NS * RPT, TAIL)])

        plsc.subcore_barrier()

        for r in range(REM):
            pltpu.sync_copy(src_hbm.at[wid, r], srcb[0])
            pltpu.sync_copy(dst_hbm.at[wid, r], dstb[0])
            pltpu.async_copy(h_hbm.at[srcb[0]], rows[0], sem_g.at[0]).wait()
            pltpu.async_copy(rows[0], acc_sh.at[dstb[0]], sem_s.at[0],
                             add=True).wait()

        for b in range(NBUF):
            pltpu.async_copy(src_hbm.at[wid, REM + b], srcb[b], sem_is.at[b])
            pltpu.async_copy(dst_hbm.at[wid, REM + b], dstb[b], sem_id.at[b])
        for b in range(NBUF):
            pltpu.make_async_copy(src_hbm.at[wid, REM + b], srcb[b],
                                  sem_is.at[b]).wait()
            pltpu.async_copy(h_hbm.at[srcb[b]], rows[b], sem_g.at[b])

        def outer(jo, carry):
            for b in range(NBUF):
                j = REM + jo * NBUF + b
                pltpu.make_async_copy(h_hbm.at[srcb[b]], rows[b],
                                      sem_g.at[b]).wait()
                pltpu.make_async_copy(dst_hbm.at[wid, j], dstb[b],
                                      sem_id.at[b]).wait()
                pltpu.async_copy(rows[b], acc_sh.at[dstb[b]],
                                 sem_s.at[b], add=True)

            @pl.when(jo < NOUT - 1)
            def _():
                jn0 = REM + (jo + 1) * NBUF
                for b in range(NBUF):
                    pltpu.async_copy(src_hbm.at[wid, jn0 + b], srcb[b],
                                     sem_is.at[b])
                for b in range(NBUF):
                    pltpu.make_async_copy(rows[b], acc_sh.at[dstb[b]],
                                          sem_s.at[b]).wait()
                    pltpu.async_copy(dst_hbm.at[wid, jn0 + b], dstb[b],
                                     sem_id.at[b])
                for b in range(NBUF):
                    pltpu.make_async_copy(src_hbm.at[wid, jn0 + b], srcb[b],
                                          sem_is.at[b]).wait()
                    pltpu.async_copy(h_hbm.at[srcb[b]], rows[b], sem_g.at[b])

            @pl.when(jo == NOUT - 1)
            def _():
                for b in range(NBUF):
                    pltpu.make_async_copy(rows[b], acc_sh.at[dstb[b]],
                                          sem_s.at[b]).wait()

            return carry

        lax.fori_loop(0, NOUT, outer, 0)

        plsc.subcore_barrier()

        pltpu.sync_copy(acc_sh.at[pl.ds(s * RPT, RPT)],
                        out_hbm.at[c, pl.ds(s * RPT, RPT)])

        @pl.when(s == NS - 1)
        def _():
            pltpu.sync_copy(acc_sh.at[pl.ds(NS * RPT, TAIL)],
                            out_hbm.at[c, pl.ds(NS * RPT, TAIL)])

    return k(h, src3, dst3)


def _mlp_chain(hprev, p0, p1, Ws, bs, flags, rows):
    grid = (N // rows,)
    nw = len(Ws)

    def body(*refs):
        x_ref, p0_ref, p1_ref = refs[0], refs[1], refs[2]
        w_refs = refs[3:3 + nw]
        b_refs = refs[3 + nw:3 + 2 * nw]
        o_ref = refs[-1]
        hloc = p0_ref[...] + p1_ref[...] - x_ref[...]
        kk = 0
        for g in range(nw):
            for i in range(w_refs[g].shape[0]):
                hloc = jnp.dot(hloc, w_refs[g][i],
                               preferred_element_type=jnp.float32)
                hloc = hloc + b_refs[g][i]
                if flags[kk]:
                    hloc = jnp.maximum(hloc, 0.0)
                kk += 1
        o_ref[...] = hloc

    row_spec = pl.BlockSpec((rows, D), lambda i: (i, 0))
    w_specs = [pl.BlockSpec(W.shape, lambda i: (0, 0, 0)) for W in Ws]
    b_specs = [pl.BlockSpec(b.shape, lambda i: (0, 0, 0)) for b in bs]
    return pl.pallas_call(
        body,
        grid=grid,
        in_specs=[row_spec, row_spec, row_spec] + w_specs + b_specs,
        out_specs=row_spec,
        out_shape=jax.ShapeDtypeStruct((N, D), jnp.float32),
    )(hprev, p0, p1, *Ws, *bs)


def kernel(x, edge_index, batch, conv_W, conv_b, lin_W, lin_b):
    src = edge_index[0].reshape(NW, NCH, CHUNK)
    dst = edge_index[1].reshape(NW, NCH, CHUNK)
    x = x.astype(jnp.float32)

    parts = _sc_scatter(x, src, dst)
    h = _mlp_chain(x, parts[0], parts[1],
                   [conv_W[0]], [conv_b[0].reshape(2, 1, D)],
                   flags=(True, True), rows=2000)

    parts = _sc_scatter(h, src, dst)
    flags = (True, False) + (True,) * 8 + (True,) * 7 + (False,)
    out = _mlp_chain(h, parts[0], parts[1],
                     [conv_W[1], lin_W.reshape(16, D, D)],
                     [conv_b[1].reshape(2, 1, D), lin_b.reshape(16, 1, D)],
                     flags=flags, rows=2000)
    return out
